# baseline (speedup 1.0000x reference)
import functools

import jax
import jax.numpy as jnp
from jax import lax
from jax.experimental import pallas as pl
from jax.experimental.pallas import tpu as pltpu
from jax.experimental.pallas import tpu_sc as plsc

SEQ = 577
NUM_HEADS = 12
NB_R = 16
R_BLK = 40
ROWS_PAD = NB_R * R_BLK
SP = 592
NCOL = SP // 16
HEAD_HALVES = 2
HEADS_PER_HALF = NUM_HEADS // HEAD_HALVES
SLAB = R_BLK * SP


def _sc_gather_body(nrd, table_hbm, idx_hbm, pos_hbm,
                    table_v, idx_v, out_v0, out_v1, tsem, isem, osem0, osem1):
    out_bufs = (out_v0, out_v1)
    osems = (osem0, osem1)
    wid = lax.axis_index("s") * 2 + lax.axis_index("c")
    hh = wid // NB_R
    rb = wid % NB_R
    r0 = rb * R_BLK
    h0 = hh * HEADS_PER_HALF

    tcopy = pltpu.make_async_copy(table_hbm, table_v, tsem)
    tcopy.start()
    icopy = pltpu.make_async_copy(
        idx_hbm.at[pl.ds(rb * SLAB, SLAB)], idx_v, isem)
    icopy.start()
    tcopy.wait()
    icopy.wait()

    ocopies = [None, None]
    for dh in range(HEADS_PER_HALF):
        s = dh % 2
        if ocopies[s] is not None:
            ocopies[s].wait()
        out_v = out_bufs[s]
        hoff = (h0 + dh) * nrd

        def row_body(r, _, out_v=out_v, hoff=hoff):
            lanes = lax.iota(jnp.int32, 16) * NCOL
            rbase = r * SP
            for c in range(NCOL):
                idx16 = idx_v[pl.ds(rbase + c * 16, 16)]
                g = plsc.load_gather(table_v, [idx16 + hoff])
                plsc.store_scatter(out_v, [lanes + (rbase + c)], g)
            return 0

        lax.fori_loop(0, R_BLK, row_body, 0)

        ocopies[s] = pltpu.make_async_copy(
            out_v,
            pos_hbm.at[pl.ds(((h0 + dh) * NB_R + rb) * SLAB, SLAB)],
            osems[s])
        ocopies[s].start()

    for s in range(2):
        if ocopies[s] is not None:
            ocopies[s].wait()


def _sc_gather(table_t_flat, idx_pad, nrd):
    mesh = plsc.VectorSubcoreMesh(core_axis_name="c", subcore_axis_name="s")
    fn = functools.partial(
        pl.kernel,
        mesh=mesh,
        out_type=jax.ShapeDtypeStruct((NUM_HEADS * ROWS_PAD * SP,), jnp.float32),
        scratch_types=[
            pltpu.VMEM((NUM_HEADS * nrd,), jnp.float32),
            pltpu.VMEM((SLAB,), jnp.int32),
            pltpu.VMEM((SLAB,), jnp.float32),
            pltpu.VMEM((SLAB,), jnp.float32),
            pltpu.SemaphoreType.DMA,
            pltpu.SemaphoreType.DMA,
            pltpu.SemaphoreType.DMA,
            pltpu.SemaphoreType.DMA,
        ],
        compiler_params=pltpu.CompilerParams(needs_layout_passes=False),
    )(functools.partial(_sc_gather_body, nrd))
    return fn(table_t_flat, idx_pad)


def _add_body(a_ref, p_ref, o_ref):
    o_ref[...] = a_ref[...] + p_ref[:, :SEQ, :SEQ][None]


def _tc_add(attn, pos_pad):
    b, nh, s, _ = attn.shape
    return pl.pallas_call(
        _add_body,
        grid=(nh,),
        in_specs=[
            pl.BlockSpec((b, 1, s, s), lambda h: (0, h, 0, 0)),
            pl.BlockSpec((1, SEQ + 7, SP), lambda h: (h, 0, 0)),
        ],
        out_specs=pl.BlockSpec((b, 1, s, s), lambda h: (0, h, 0, 0)),
        out_shape=jax.ShapeDtypeStruct(attn.shape, attn.dtype),
        compiler_params=pltpu.CompilerParams(
            vmem_limit_bytes=110 * 1024 * 1024,
        ),
    )(attn, pos_pad)


def kernel(attention_scores, relative_position_bias_table, relative_position_index):
    nrd = relative_position_bias_table.shape[0]
    table_t_flat = jnp.transpose(relative_position_bias_table).reshape(-1)
    idx_pad = jnp.pad(
        relative_position_index,
        ((0, ROWS_PAD - SEQ), (0, SP - SEQ)),
    ).reshape(ROWS_PAD, 16, NCOL).swapaxes(1, 2).reshape(-1)
    pos_flat = _sc_gather(table_t_flat, idx_pad, nrd)
    pos_pad = pos_flat.reshape(NUM_HEADS, ROWS_PAD, SP)
    return _tc_add(attention_scores, pos_pad)
